# baseline (speedup 1.0000x reference)
import functools

import jax
import jax.numpy as jnp
from jax import lax
from jax.experimental import pallas as pl
from jax.experimental.pallas import tpu as pltpu
from jax.experimental.pallas import tpu_sc as plsc

B, H, W = 4, 224, 224
F = 192
L = 16
NG = F // L
NC, NS = 2, 16
NW = NC * NS
P = 56
NCHUNK = W // P
ROWS_PER_W = (B * H) // NW
T = ROWS_PER_W * NCHUNK


NP = NG // 2


def _index_table():
    m = jnp.arange(F, dtype=jnp.int32)
    start = m.astype(jnp.float32)
    stop = (m + F - 1).astype(jnp.float32)
    idx = jnp.linspace(start, stop, F, axis=-1)
    idx = idx.astype(jnp.int32)
    idx = jnp.mod(idx, F)
    r = idx.reshape(F, NP, 2, L)
    packed = r[:, :, 0, :] | (r[:, :, 1, :] << 16)
    return packed.reshape(F * NP * L)


def _make_roll():
    mesh = plsc.VectorSubcoreMesh(core_axis_name="c", subcore_axis_name="s")

    @functools.partial(
        pl.kernel,
        out_type=jax.ShapeDtypeStruct((B, H, W, F), jnp.float32),
        mesh=mesh,
        compiler_params=pltpu.CompilerParams(
            needs_layout_passes=False, use_tc_tiling_on_sc=True),
        scratch_types=[
            pltpu.VMEM((F * NP * L,), jnp.int32),
            pltpu.VMEM((2, P, F), jnp.float32),
            pltpu.VMEM((2, P, F), jnp.float32),
            pltpu.VMEM((2, P), jnp.int32),
            pltpu.SemaphoreType.DMA,
            pltpu.SemaphoreType.DMA,
            pltpu.SemaphoreType.DMA,
            pltpu.SemaphoreType.DMA,
            pltpu.SemaphoreType.DMA,
            pltpu.SemaphoreType.DMA,
        ],
    )
    def roll(x_hbm, m_hbm, tbl_hbm, out_hbm, tbl, xv, ov, mv,
             sx0, sx1, so0, so1, sm0, sm1):
        wid = lax.axis_index("s") * NC + lax.axis_index("c")
        b = lax.shift_right_logical(wid, 3)
        h0 = (wid & 7) * ROWS_PER_W
        pltpu.sync_copy(tbl_hbm, tbl)
        sx = (sx0, sx1)
        so = (so0, so1)
        sm = (sm0, sm1)
        iota = lax.broadcasted_iota(jnp.int32, (L,), 0)

        def loc(tt):
            h = h0 + lax.shift_right_logical(tt, 2)
            w0 = (tt & 3) * P
            pix0 = (b * H + h) * W + w0
            return h, w0, pix0

        def in_copies(tt, bi):
            h, w0, pix0 = loc(tt)
            cx = pltpu.make_async_copy(
                x_hbm.at[b, h, pl.ds(w0, P)], xv.at[bi], sx[bi])
            cm = pltpu.make_async_copy(
                m_hbm.at[pl.ds(pix0, P)], mv.at[bi], sm[bi])
            return cx, cm

        def out_copy(tt, bi):
            h, w0, _ = loc(tt)
            return pltpu.make_async_copy(
                ov.at[bi], out_hbm.at[b, h, pl.ds(w0, P)], so[bi])

        for bi in range(2):
            cx, cm = in_copies(bi, bi)
            cx.start()
            cm.start()

        def step(i, carry):
            for bi in range(2):
                tt = 2 * i + bi
                cx, cm = in_copies(tt, bi)
                cx.wait()
                cm.wait()

                @pl.when(tt >= 2)
                def _():
                    out_copy(tt - 2, bi).wait()

                xb = xv.at[bi]
                ob = ov.at[bi]
                mb_ref = mv.at[bi]

                @plsc.parallel_loop(0, P, step=1, unroll=8)
                def pix_body(q):
                    qvec = jnp.broadcast_to(q, (L,))
                    mq = plsc.load_gather(mb_ref, [qvec])
                    trow = mq * (NP * L)
                    for p in range(NP):
                        tv = plsc.load_gather(tbl, [trow + (iota + p * L)])
                        ch0 = tv & 0xFFFF
                        ch1 = lax.shift_right_logical(tv, 16)
                        v0 = plsc.load_gather(xb, [qvec, ch0])
                        ob[q, pl.ds(2 * p * L, L)] = v0
                        v1 = plsc.load_gather(xb, [qvec, ch1])
                        ob[q, pl.ds((2 * p + 1) * L, L)] = v1

                out_copy(tt, bi).start()

                @pl.when(tt + 2 < T)
                def _():
                    cx2, cm2 = in_copies(tt + 2, bi)
                    cx2.start()
                    cm2.start()
            return carry

        lax.fori_loop(0, T // 2, step, 0)
        out_copy(T - 2, 0).wait()
        out_copy(T - 1, 1).wait()

    return roll


def kernel(x, map):
    n = B * H * W
    mf = map.reshape(n)
    tbl = _index_table()
    return _make_roll()(x, mf, tbl)


# speedup vs baseline: 1.0894x; 1.0894x over previous
import functools

import jax
import jax.numpy as jnp
from jax import lax
from jax.experimental import pallas as pl
from jax.experimental.pallas import tpu as pltpu
from jax.experimental.pallas import tpu_sc as plsc

B, H, W = 4, 224, 224
F = 192
L = 16
NG = F // L
NC, NS = 2, 16
NW = NC * NS
P = 112
NCHUNK = W // P
ROWS_PER_W = (B * H) // NW
T = ROWS_PER_W * NCHUNK


NP = NG // 4


def _index_table():
    m = jnp.arange(F, dtype=jnp.int32)
    start = m.astype(jnp.float32)
    stop = (m + F - 1).astype(jnp.float32)
    idx = jnp.linspace(start, stop, F, axis=-1)
    idx = idx.astype(jnp.int32)
    idx = jnp.mod(idx, F)
    r = idx.reshape(F, NP, 4, L)
    packed = (r[:, :, 0, :] | (r[:, :, 1, :] << 8)
              | (r[:, :, 2, :] << 16) | (r[:, :, 3, :] << 24))
    return packed.reshape(F * NP * L)


def _make_roll():
    mesh = plsc.VectorSubcoreMesh(core_axis_name="c", subcore_axis_name="s")

    @functools.partial(
        pl.kernel,
        out_type=jax.ShapeDtypeStruct((B, H, W, F), jnp.float32),
        mesh=mesh,
        compiler_params=pltpu.CompilerParams(
            needs_layout_passes=False, use_tc_tiling_on_sc=True),
        scratch_types=[
            pltpu.VMEM((F * NP * L,), jnp.int32),
            pltpu.VMEM((2, P, F), jnp.float32),
            pltpu.VMEM((2, P, F), jnp.float32),
            pltpu.VMEM((2, P), jnp.int32),
            pltpu.SemaphoreType.DMA,
            pltpu.SemaphoreType.DMA,
            pltpu.SemaphoreType.DMA,
            pltpu.SemaphoreType.DMA,
            pltpu.SemaphoreType.DMA,
            pltpu.SemaphoreType.DMA,
        ],
    )
    def roll(x_hbm, m_hbm, tbl_hbm, out_hbm, tbl, xv, ov, mv,
             sx0, sx1, so0, so1, sm0, sm1):
        wid = lax.axis_index("s") * NC + lax.axis_index("c")
        b = lax.shift_right_logical(wid, 3)
        h0 = (wid & 7) * ROWS_PER_W
        pltpu.sync_copy(tbl_hbm, tbl)
        sx = (sx0, sx1)
        so = (so0, so1)
        sm = (sm0, sm1)
        iota = lax.broadcasted_iota(jnp.int32, (L,), 0)

        def loc(tt):
            h = h0 + lax.shift_right_logical(tt, 1)
            w0 = (tt & 1) * P
            pix0 = (b * H + h) * W + w0
            return h, w0, pix0

        def in_copies(tt, bi):
            h, w0, pix0 = loc(tt)
            cx = pltpu.make_async_copy(
                x_hbm.at[b, h, pl.ds(w0, P)], xv.at[bi], sx[bi])
            cm = pltpu.make_async_copy(
                m_hbm.at[pl.ds(pix0, P)], mv.at[bi], sm[bi])
            return cx, cm

        def out_copy(tt, bi):
            h, w0, _ = loc(tt)
            return pltpu.make_async_copy(
                ov.at[bi], out_hbm.at[b, h, pl.ds(w0, P)], so[bi])

        for bi in range(2):
            cx, cm = in_copies(bi, bi)
            cx.start()
            cm.start()

        def step(i, carry):
            for bi in range(2):
                tt = 2 * i + bi
                cx, cm = in_copies(tt, bi)
                cx.wait()
                cm.wait()

                @pl.when(tt >= 2)
                def _():
                    out_copy(tt - 2, bi).wait()

                xb = xv.at[bi]
                ob = ov.at[bi]
                mb_ref = mv.at[bi]

                @plsc.parallel_loop(0, P, step=1, unroll=4)
                def pix_body(q):
                    qvec = jnp.broadcast_to(q, (L,))
                    mq = plsc.load_gather(mb_ref, [qvec])
                    trow = mq * (NP * L)
                    for p in range(NP):
                        tv = plsc.load_gather(tbl, [trow + (iota + p * L)])
                        for s in range(4):
                            ch = lax.shift_right_logical(tv, 8 * s) if s else tv
                            ch = ch & 0xFF if s < 3 else ch
                            val = plsc.load_gather(xb, [qvec, ch])
                            ob[q, pl.ds((4 * p + s) * L, L)] = val

                out_copy(tt, bi).start()

                @pl.when(tt + 2 < T)
                def _():
                    cx2, cm2 = in_copies(tt + 2, bi)
                    cx2.start()
                    cm2.start()
            return carry

        lax.fori_loop(0, T // 2, step, 0)
        out_copy(T - 2, 0).wait()
        out_copy(T - 1, 1).wait()

    return roll


def kernel(x, map):
    n = B * H * W
    mf = map.reshape(n)
    tbl = _index_table()
    return _make_roll()(x, mf, tbl)
